# baseline (speedup 1.0000x reference)
import functools

import jax
import jax.numpy as jnp
from jax.experimental import pallas as pl

_N = 10000
_E = 320000
_H = 128
_ROW_BLK = 2000
_EDGE_BLK = 4000


def _layer_tc_body(agg_ref, nd_ref, w_ref, b_ref, ns_ref, h_ref, m_ref):
    agg = agg_ref[...] * nd_ref[...]
    h = jnp.maximum(
        jax.lax.dot_general(agg.astype(jnp.bfloat16), w_ref[...].astype(jnp.bfloat16),
                            (((1,), (0,)), ((), ())),
                            preferred_element_type=jnp.float32) + b_ref[...], 0.0)
    h_ref[...] = h
    m_ref[...] = h * ns_ref[...]


def _layer_tc(agg, norm_dst, W, b, norm_src):
    grid = (_N // _ROW_BLK,)
    return pl.pallas_call(
        _layer_tc_body,
        grid=grid,
        in_specs=[
            pl.BlockSpec((_ROW_BLK, _H), lambda i: (i, 0)),
            pl.BlockSpec((_ROW_BLK, 1), lambda i: (i, 0)),
            pl.BlockSpec((_H, _H), lambda i: (0, 0)),
            pl.BlockSpec((1, _H), lambda i: (0, 0)),
            pl.BlockSpec((_ROW_BLK, 1), lambda i: (i, 0)),
        ],
        out_specs=[
            pl.BlockSpec((_ROW_BLK, _H), lambda i: (i, 0)),
            pl.BlockSpec((_ROW_BLK, _H), lambda i: (i, 0)),
        ],
        out_shape=[
            jax.ShapeDtypeStruct((_N, _H), jnp.float32),
            jax.ShapeDtypeStruct((_N, _H), jnp.float32),
        ],
    )(agg, norm_dst.reshape(_N, 1), W, b.reshape(1, _H), norm_src.reshape(_N, 1))


def _edge_msg_body(hs_ref, hd_ref, tw_ref, pw_ref, tb_ref, pb_ref,
                   msg_ref, s1_ref, s2_ref):
    i = pl.program_id(0)
    hs = hs_ref[...]
    hd = hd_ref[...]
    d = (hs - hd).astype(jnp.bfloat16)
    m1 = jax.lax.dot_general(d, tw_ref[...].astype(jnp.bfloat16),
                             (((1,), (0,)), ((), ())),
                             preferred_element_type=jnp.float32)
    m2 = jax.lax.dot_general(hd.astype(jnp.bfloat16), pw_ref[...].astype(jnp.bfloat16),
                             (((1,), (0,)), ((), ())),
                             preferred_element_type=jnp.float32)
    msg = (m1 + tb_ref[...]) + m2 + pb_ref[...]
    msg_ref[...] = msg

    @pl.when(i == 0)
    def _init():
        s1_ref[...] = jnp.zeros_like(s1_ref)
        s2_ref[...] = jnp.zeros_like(s2_ref)

    s1_ref[...] += jnp.sum(msg, axis=0, keepdims=True)
    s2_ref[...] += jnp.sum(msg * msg, axis=0, keepdims=True)


def _edge_msg(h_src, h_dst, theta_w, phi_w, theta_b, phi_b):
    grid = (_E // _EDGE_BLK,)
    return pl.pallas_call(
        _edge_msg_body,
        grid=grid,
        in_specs=[
            pl.BlockSpec((_EDGE_BLK, _H), lambda i: (i, 0)),
            pl.BlockSpec((_EDGE_BLK, _H), lambda i: (i, 0)),
            pl.BlockSpec((_H, _H), lambda i: (0, 0)),
            pl.BlockSpec((_H, _H), lambda i: (0, 0)),
            pl.BlockSpec((1, _H), lambda i: (0, 0)),
            pl.BlockSpec((1, _H), lambda i: (0, 0)),
        ],
        out_specs=[
            pl.BlockSpec((_EDGE_BLK, _H), lambda i: (i, 0)),
            pl.BlockSpec((1, _H), lambda i: (0, 0)),
            pl.BlockSpec((1, _H), lambda i: (0, 0)),
        ],
        out_shape=[
            jax.ShapeDtypeStruct((_E, _H), jnp.float32),
            jax.ShapeDtypeStruct((1, _H), jnp.float32),
            jax.ShapeDtypeStruct((1, _H), jnp.float32),
        ],
    )(h_src, h_dst, theta_w, phi_w, theta_b.reshape(1, _H), phi_b.reshape(1, _H))


def kernel(node_feat, edge_index, edge_feat, W1, b1, W2, b2, W3, b3,
           theta_w, theta_b, phi_w, phi_b, gamma, beta):
    src = edge_index[0]
    dst = edge_index[1]
    ones = jnp.ones((_E,), jnp.float32)
    deg_out = jax.ops.segment_sum(ones, src, num_segments=_N)
    deg_in = jax.ops.segment_sum(ones, dst, num_segments=_N)
    norm_src = 1.0 / jnp.sqrt(jnp.clip(deg_out, 1.0))
    norm_dst = 1.0 / jnp.sqrt(jnp.clip(deg_in, 1.0))

    m = node_feat * norm_src[:, None]
    h = None
    for W, b in ((W1, b1), (W2, b2), (W3, b3)):
        agg = jax.ops.segment_sum(jnp.take(m, src, axis=0), dst, num_segments=_N)
        h, m = _layer_tc(agg, norm_dst, W, b, norm_src)

    h_src = jnp.take(h, src, axis=0)
    h_dst = jnp.take(h, dst, axis=0)
    msg, s1, s2 = _edge_msg(h_src, h_dst, theta_w, phi_w, theta_b, phi_b)
    mean = s1[0] / _E
    var = s2[0] / _E - mean * mean
    inv_std = 1.0 / jnp.sqrt(var + 1e-5)
    M = jax.ops.segment_max(msg, dst, num_segments=_N)
    out = (M - mean) * inv_std * gamma + beta
    return jnp.where(deg_in[:, None] > 0, out, 0.0)
